# baseline (speedup 1.0000x reference)
import functools
import jax
import jax.numpy as jnp
from jax import lax
from jax.experimental import pallas as pl
from jax.experimental.pallas import tpu as pltpu
from jax.experimental.pallas import tpu_sc as plsc

_L = 16


def _make_sc_add(B, T, D):
    info = plsc.get_sparse_core_info()
    NC, NS = info.num_cores, info.num_subcores
    NW = NC * NS
    rows = B * T
    rows_per_w = rows // NW
    chunk_rows = 64
    n_chunks = rows_per_w // chunk_rows
    CH = chunk_rows * D
    n_vec = CH // _L

    mesh = plsc.VectorSubcoreMesh(core_axis_name="c", subcore_axis_name="s")

    @functools.partial(
        pl.kernel,
        mesh=mesh,
        out_type=jax.ShapeDtypeStruct((rows * D,), jnp.float32),
        scratch_types=[
            pltpu.VMEM((CH,), jnp.float32),
            pltpu.VMEM((CH,), jnp.float32),
        ],
    )
    def sc_add(x_hbm, pos_hbm, out_hbm, xv, pv):
        wid = lax.axis_index("s") * NC + lax.axis_index("c")
        row0 = wid * rows_per_w
        prow0 = lax.rem(row0, T)

        def chunk_body(c, _):
            xoff = (row0 + c * chunk_rows) * D
            poff = (prow0 + c * chunk_rows) * D
            pltpu.sync_copy(x_hbm.at[pl.ds(xoff, CH)], xv)
            pltpu.sync_copy(pos_hbm.at[pl.ds(poff, CH)], pv)

            def vec_body(i, _):
                sl = pl.ds(i * _L, _L)
                xv[sl] = xv[sl] + pv[sl]
                return ()

            lax.fori_loop(0, n_vec, vec_body, (), unroll=16)
            pltpu.sync_copy(xv, out_hbm.at[pl.ds(xoff, CH)])
            return ()

        lax.fori_loop(0, n_chunks, chunk_body, ())

    return sc_add


def kernel(x, pos_table):
    T, D = pos_table.shape
    xr = x.reshape(-1, T, D)
    B = xr.shape[0]
    sc_add = _make_sc_add(B, T, D)
    out = sc_add(xr.reshape(-1), pos_table.reshape(-1))
    return out.reshape(B, T, D)


# speedup vs baseline: 1.3506x; 1.3506x over previous
import functools
import jax
import jax.numpy as jnp
from jax import lax
from jax.experimental import pallas as pl
from jax.experimental.pallas import tpu as pltpu
from jax.experimental.pallas import tpu_sc as plsc

_L = 16


def _make_sc_add(B, T, D):
    info = plsc.get_sparse_core_info()
    NC, NS = info.num_cores, info.num_subcores
    NW = NC * NS
    rows = B * T
    rows_per_w = rows // NW
    chunk_rows = 64
    n_chunks = rows_per_w // chunk_rows
    CH = chunk_rows * D
    n_vec = CH // _L

    mesh = plsc.VectorSubcoreMesh(core_axis_name="c", subcore_axis_name="s")

    @functools.partial(
        pl.kernel,
        mesh=mesh,
        out_type=jax.ShapeDtypeStruct((rows * D,), jnp.float32),
        scratch_types=[
            pltpu.VMEM((CH,), jnp.float32),
            pltpu.VMEM((CH,), jnp.float32),
        ],
    )
    def sc_add(x_hbm, pos_hbm, out_hbm, xv, pv):
        wid = lax.axis_index("s") * NC + lax.axis_index("c")
        row0 = wid * rows_per_w
        prow0 = lax.rem(row0, T)

        def chunk_body(c, _):
            xoff = (row0 + c * chunk_rows) * D
            poff = (prow0 + c * chunk_rows) * D
            pltpu.sync_copy(x_hbm.at[pl.ds(xoff, CH)], xv)
            pltpu.sync_copy(pos_hbm.at[pl.ds(poff, CH)], pv)

            U = 16

            def vec_body(i, _):
                base = i * (_L * U)
                for j in range(U):
                    sl = pl.ds(base + j * _L, _L)
                    xv[sl] = xv[sl] + pv[sl]
                return ()

            lax.fori_loop(0, n_vec // U, vec_body, ())
            pltpu.sync_copy(xv, out_hbm.at[pl.ds(xoff, CH)])
            return ()

        lax.fori_loop(0, n_chunks, chunk_body, ())

    return sc_add


def kernel(x, pos_table):
    T, D = pos_table.shape
    xr = x.reshape(-1, T, D)
    B = xr.shape[0]
    sc_add = _make_sc_add(B, T, D)
    out = sc_add(xr.reshape(-1), pos_table.reshape(-1))
    return out.reshape(B, T, D)


# speedup vs baseline: 8.1923x; 6.0656x over previous
import jax
import jax.numpy as jnp
from jax.experimental import pallas as pl

_BR = 1024


def _add_body(x_ref, *refs):
    o_ref = refs[-1]
    p_refs = refs[:-1]
    K = len(p_refs)
    i = pl.program_id(0)
    for k in range(K):
        @pl.when(i % K == k)
        def _(k=k):
            o_ref[...] = x_ref[...] + p_refs[k][...]


def kernel(x, pos_table):
    T, D = pos_table.shape
    xf = x.reshape(-1, D)
    N = xf.shape[0]
    K = T // _BR
    grid = (N // _BR,)
    out = pl.pallas_call(
        _add_body,
        grid=grid,
        in_specs=[pl.BlockSpec((_BR, D), lambda i: (i, 0))]
        + [pl.BlockSpec((_BR, D), lambda i, k=k: (k, 0)) for k in range(K)],
        out_specs=pl.BlockSpec((_BR, D), lambda i: (i, 0)),
        out_shape=jax.ShapeDtypeStruct((N, D), x.dtype),
    )(xf, *([pos_table] * K))
    return out.reshape(-1, T, D)


# speedup vs baseline: 9.4622x; 1.1550x over previous
import jax
import jax.numpy as jnp
from jax.experimental import pallas as pl


def _add_body(x_ref, p_ref, o_ref):
    T = p_ref.shape[0]
    o_ref[:T] = x_ref[:T] + p_ref[...]
    o_ref[T:] = x_ref[T:] + p_ref[...]


def kernel(x, pos_table):
    T, D = pos_table.shape
    xf = x.reshape(-1, D)
    N = xf.shape[0]
    BR = 2 * T
    grid = (N // BR,)
    out = pl.pallas_call(
        _add_body,
        grid=grid,
        in_specs=[
            pl.BlockSpec((BR, D), lambda i: (i, 0)),
            pl.BlockSpec((T, D), lambda i: (0, 0)),
        ],
        out_specs=pl.BlockSpec((BR, D), lambda i: (i, 0)),
        out_shape=jax.ShapeDtypeStruct((N, D), x.dtype),
    )(xf, pos_table)
    return out.reshape(-1, T, D)
